# baseline (speedup 1.0000x reference)
import functools

import jax
import jax.numpy as jnp
from jax import lax
from jax.experimental import pallas as pl
from jax.experimental.pallas import tpu as pltpu
from jax.experimental.pallas import tpu_sc as plsc

F_FIELDS = 26
VOCAB = 100000
D = 64
B = 1024
IN_DIM = F_FIELDS * D
N_ROWS = B * F_FIELDS

_NC = 2
_NS = 16
_NW = _NC * _NS
_RPW = N_ROWS // _NW
_CHUNK = 104
_NCHUNK = _RPW // _CHUNK

def _sc_gather_body(idx_hbm, table_hbm, out_hbm, idx_v, rows_v, sem):
    wid = lax.axis_index("s") * _NC + lax.axis_index("c")
    base = wid * _RPW

    pltpu.sync_copy(idx_hbm.at[pl.ds(base, _RPW)], idx_v)

    def _add_offset(j, carry):
        pos = base + j * 16 + lax.iota(jnp.int32, 16)
        f = lax.rem(pos, F_FIELDS)
        sl = pl.ds(j * 16, 16)
        idx_v[sl] = idx_v[sl] + f * VOCAB
        return carry

    lax.fori_loop(0, _RPW // 16, _add_offset, 0)

    copies = []
    for c in range(_NCHUNK):
        cp = pltpu.make_async_copy(
            table_hbm.at[idx_v.at[pl.ds(c * _CHUNK, _CHUNK)]],
            rows_v.at[pl.ds(c * _CHUNK, _CHUNK)],
            sem,
        )
        cp.start()
        copies.append(cp)
    for cp in copies:
        cp.wait()

    pltpu.sync_copy(rows_v, out_hbm.at[pl.ds(base, _RPW)])


@functools.lru_cache(maxsize=1)
def _make_sc_gather():
    mesh = plsc.VectorSubcoreMesh(
        core_axis_name="c", subcore_axis_name="s", num_cores=_NC)
    return pl.kernel(
        _sc_gather_body,
        mesh=mesh,
        compiler_params=pltpu.CompilerParams(use_tc_tiling_on_sc=False),
        out_type=jax.ShapeDtypeStruct((N_ROWS, D), jnp.float32),
        scratch_types=[
            pltpu.VMEM((_RPW,), jnp.int32),
            pltpu.VMEM((_RPW, D), jnp.float32),
            pltpu.SemaphoreType.DMA,
        ],
    )


def _tc_body(c_ref, wk_ref, w1_ref, b1_ref, w2_ref, b2_ref, w3_ref, b3_ref,
             w4_ref, b4_ref, o_ref):
    c = c_ref[...]
    z = jnp.dot(c, wk_ref[...], preferred_element_type=jnp.float32)
    fw = 0.5 * jnp.sum(c * z, axis=1, keepdims=True)
    h = jnp.maximum(
        jnp.dot(c, w1_ref[...], preferred_element_type=jnp.float32)
        + b1_ref[...], 0.0)
    h = jnp.maximum(
        jnp.dot(h, w2_ref[...], preferred_element_type=jnp.float32)
        + b2_ref[...], 0.0)
    h = jnp.maximum(
        jnp.dot(h, w3_ref[...], preferred_element_type=jnp.float32)
        + b3_ref[...], 0.0)
    m = jnp.dot(h, w4_ref[...], preferred_element_type=jnp.float32) + b4_ref[...]
    o_ref[...] = jax.nn.sigmoid(fw + m)


_BB = 256


def _tc_call(c, wk, W1, b1, W2, b2, W3, b3, W4, b4):
    full = lambda i: (0, 0)
    return pl.pallas_call(
        _tc_body,
        grid=(B // _BB,),
        in_specs=[
            pl.BlockSpec((_BB, IN_DIM), lambda i: (i, 0)),
            pl.BlockSpec((IN_DIM, IN_DIM), full),
            pl.BlockSpec((IN_DIM, 512), full),
            pl.BlockSpec((1, 512), full),
            pl.BlockSpec((512, 256), full),
            pl.BlockSpec((1, 256), full),
            pl.BlockSpec((256, 128), full),
            pl.BlockSpec((1, 128), full),
            pl.BlockSpec((128, 1), full),
            pl.BlockSpec((1, 1), full),
        ],
        out_specs=pl.BlockSpec((_BB, 1), lambda i: (i, 0)),
        out_shape=jax.ShapeDtypeStruct((B, 1), jnp.float32),
    )(c, wk, W1, b1, W2, b2, W3, b3, W4, b4)


def kernel(x, tables, field_cov_W, W1, b1, W2, b2, W3, b3, W4, b4):
    idx_raw = x.astype(jnp.int32).reshape(-1)
    table2d = tables.reshape(F_FIELDS * VOCAB, D)
    gathered = _make_sc_gather()(idx_raw, table2d)
    c = gathered.reshape(B, IN_DIM)

    w_sym = (field_cov_W + field_cov_W.T) * 0.5
    w0 = w_sym * (1.0 - jnp.eye(F_FIELDS, dtype=jnp.float32))
    wk = (w0[:, None, :, None]
          * jnp.eye(D, dtype=jnp.float32)[None, :, None, :]
          ).reshape(IN_DIM, IN_DIM)

    out = _tc_call(c, wk, W1, b1.reshape(1, -1), W2, b2.reshape(1, -1),
                   W3, b3.reshape(1, -1), W4, b4.reshape(1, -1))
    return out.reshape(B)


# speedup vs baseline: 1.5948x; 1.5948x over previous
import functools

import jax
import jax.numpy as jnp
from jax import lax
from jax.experimental import pallas as pl
from jax.experimental.pallas import tpu as pltpu
from jax.experimental.pallas import tpu_sc as plsc

F_FIELDS = 26
VOCAB = 100000
D = 64
B = 1024
IN_DIM = F_FIELDS * D
N_ROWS = B * F_FIELDS

_NC = 2
_NS = 16
_NW = _NC * _NS
_RPW = N_ROWS // _NW
_CHUNK = 104
_NCHUNK = _RPW // _CHUNK

def _sc_gather_body(idx_hbm, table_hbm, out_hbm, idx_v, rows_v, sem):
    wid = lax.axis_index("s") * _NC + lax.axis_index("c")
    base = wid * _RPW

    pltpu.sync_copy(idx_hbm.at[pl.ds(base, _RPW)], idx_v.at[pl.ds(0, _RPW)])

    def _start(j):
        v = idx_v[pl.ds(j, 16)][0]
        f = lax.rem(base + j, F_FIELDS)
        return pltpu.make_async_copy(
            table_hbm.at[f, pl.ds(v, 1), :],
            rows_v.at[pl.ds(j, 1), :],
            sem,
        )

    _WIN = 16

    def _prime(j, carry):
        _start(j).start()
        return carry

    lax.fori_loop(0, _WIN, _prime, 0)

    def _step(j, carry):
        _start(j + _WIN).start()
        _start(j).wait()
        return carry

    lax.fori_loop(0, _RPW - _WIN, _step, 0)

    def _drain(j, carry):
        _start(j).wait()
        return carry

    lax.fori_loop(_RPW - _WIN, _RPW, _drain, 0)

    pltpu.sync_copy(rows_v, out_hbm.at[pl.ds(base, _RPW)])


@functools.lru_cache(maxsize=1)
def _make_sc_gather():
    mesh = plsc.VectorSubcoreMesh(
        core_axis_name="c", subcore_axis_name="s", num_cores=_NC)
    return pl.kernel(
        _sc_gather_body,
        mesh=mesh,
        out_type=jax.ShapeDtypeStruct((N_ROWS, D), jnp.float32),
        scratch_types=[
            pltpu.VMEM((_RPW + 16,), jnp.int32),
            pltpu.VMEM((_RPW, D), jnp.float32),
            pltpu.SemaphoreType.DMA,
        ],
    )


def _tc_body(c_ref, wk_ref, w1_ref, b1_ref, w2_ref, b2_ref, w3_ref, b3_ref,
             w4_ref, b4_ref, o_ref):
    c = c_ref[...]
    z = jnp.dot(c, wk_ref[...], preferred_element_type=jnp.float32)
    fw = 0.5 * jnp.sum(c * z, axis=1, keepdims=True)
    h = jnp.maximum(
        jnp.dot(c, w1_ref[...], preferred_element_type=jnp.float32)
        + b1_ref[...], 0.0)
    h = jnp.maximum(
        jnp.dot(h, w2_ref[...], preferred_element_type=jnp.float32)
        + b2_ref[...], 0.0)
    h = jnp.maximum(
        jnp.dot(h, w3_ref[...], preferred_element_type=jnp.float32)
        + b3_ref[...], 0.0)
    m = jnp.dot(h, w4_ref[...], preferred_element_type=jnp.float32) + b4_ref[...]
    o_ref[...] = jax.nn.sigmoid(fw + m)


_BB = 256


def _tc_call(c, wk, W1, b1, W2, b2, W3, b3, W4, b4):
    full = lambda i: (0, 0)
    return pl.pallas_call(
        _tc_body,
        grid=(B // _BB,),
        in_specs=[
            pl.BlockSpec((_BB, IN_DIM), lambda i: (i, 0)),
            pl.BlockSpec((IN_DIM, IN_DIM), full),
            pl.BlockSpec((IN_DIM, 512), full),
            pl.BlockSpec((1, 512), full),
            pl.BlockSpec((512, 256), full),
            pl.BlockSpec((1, 256), full),
            pl.BlockSpec((256, 128), full),
            pl.BlockSpec((1, 128), full),
            pl.BlockSpec((128, 1), full),
            pl.BlockSpec((1, 1), full),
        ],
        out_specs=pl.BlockSpec((_BB, 1), lambda i: (i, 0)),
        out_shape=jax.ShapeDtypeStruct((B, 1), jnp.float32),
    )(c, wk, W1, b1, W2, b2, W3, b3, W4, b4)


def kernel(x, tables, field_cov_W, W1, b1, W2, b2, W3, b3, W4, b4):
    idx_raw = x.astype(jnp.int32).reshape(-1)
    gathered = _make_sc_gather()(idx_raw, tables)
    c = gathered.reshape(B, IN_DIM)

    w_sym = (field_cov_W + field_cov_W.T) * 0.5
    w0 = w_sym * (1.0 - jnp.eye(F_FIELDS, dtype=jnp.float32))
    wk = (w0[:, None, :, None]
          * jnp.eye(D, dtype=jnp.float32)[None, :, None, :]
          ).reshape(IN_DIM, IN_DIM)

    out = _tc_call(c, wk, W1, b1.reshape(1, -1), W2, b2.reshape(1, -1),
                   W3, b3.reshape(1, -1), W4, b4.reshape(1, -1))
    return out.reshape(B)


# speedup vs baseline: 3.3914x; 2.1265x over previous
import functools

import jax
import jax.numpy as jnp
from jax import lax
from jax.experimental import pallas as pl
from jax.experimental.pallas import tpu as pltpu
from jax.experimental.pallas import tpu_sc as plsc

F_FIELDS = 26
VOCAB = 100000
D = 64
B = 1024
IN_DIM = F_FIELDS * D
N_ROWS = B * F_FIELDS

_NC = 2
_NS = 16
_NW = _NC * _NS
_RPW = N_ROWS // _NW
_CHUNK = 104
_NCHUNK = _RPW // _CHUNK

_WIN = 8


def _sc_gather_body(idx_hbm, t3_hbm, out_hbm, idx_v, rows_v, blks, sem):
    wid = lax.axis_index("s") * _NC + lax.axis_index("c")
    base = wid * _RPW

    pltpu.sync_copy(idx_hbm.at[pl.ds(base, _RPW)], idx_v.at[pl.ds(0, _RPW)])

    def _fetch(j, w):
        v = idx_v[pl.ds(j, 16)][0]
        f = lax.rem(base + j, F_FIELDS)
        voff = pl.multiple_of((v >> 7) << 7, 128)
        pltpu.make_async_copy(
            t3_hbm.at[f, :, pl.ds(voff, 128)], blks.at[w], sem).start()

    def _extract(j, w):
        v = idx_v[pl.ds(j, 16)][0]
        lane = jnp.full((16,), v & 127, dtype=jnp.int32)
        for k in range(D // 16):
            d_idx = lax.iota(jnp.int32, 16) + (16 * k)
            col = plsc.load_gather(blks.at[w], [d_idx, lane])
            rows_v[pl.ds(j * D + 16 * k, 16)] = col

    def _group(g, carry):
        jb = g * _WIN
        for w in range(_WIN):
            _fetch(jb + w, w)
        for w in range(_WIN):
            pltpu.make_async_copy(
                t3_hbm.at[0, :, pl.ds(0, 128)], blks.at[w], sem).wait()
        for w in range(_WIN):
            _extract(jb + w, w)
        return carry

    lax.fori_loop(0, _RPW // _WIN, _group, 0)

    pltpu.sync_copy(rows_v, out_hbm.at[pl.ds(base * D, _RPW * D)])


@functools.lru_cache(maxsize=1)
def _make_sc_gather():
    mesh = plsc.VectorSubcoreMesh(
        core_axis_name="c", subcore_axis_name="s", num_cores=_NC)
    return pl.kernel(
        _sc_gather_body,
        mesh=mesh,
        compiler_params=pltpu.CompilerParams(needs_layout_passes=False),
        out_type=jax.ShapeDtypeStruct((N_ROWS * D,), jnp.float32),
        scratch_types=[
            pltpu.VMEM((_RPW + 16,), jnp.int32),
            pltpu.VMEM((_RPW * D,), jnp.float32),
            pltpu.VMEM((_WIN, D, 128), jnp.float32),
            pltpu.SemaphoreType.DMA,
        ],
    )


def _tc_body(c_ref, wk_ref, w1_ref, b1_ref, w2_ref, b2_ref, w3_ref, b3_ref,
             w4_ref, b4_ref, o_ref):
    c = c_ref[...]
    z = jnp.dot(c, wk_ref[...], preferred_element_type=jnp.float32)
    fw = 0.5 * jnp.sum(c * z, axis=1, keepdims=True)
    h = jnp.maximum(
        jnp.dot(c, w1_ref[...], preferred_element_type=jnp.float32)
        + b1_ref[...], 0.0)
    h = jnp.maximum(
        jnp.dot(h, w2_ref[...], preferred_element_type=jnp.float32)
        + b2_ref[...], 0.0)
    h = jnp.maximum(
        jnp.dot(h, w3_ref[...], preferred_element_type=jnp.float32)
        + b3_ref[...], 0.0)
    m = jnp.dot(h, w4_ref[...], preferred_element_type=jnp.float32) + b4_ref[...]
    o_ref[...] = jax.nn.sigmoid(fw + m)


_BB = 256


def _tc_call(c, wk, W1, b1, W2, b2, W3, b3, W4, b4):
    full = lambda i: (0, 0)
    return pl.pallas_call(
        _tc_body,
        grid=(B // _BB,),
        in_specs=[
            pl.BlockSpec((_BB, IN_DIM), lambda i: (i, 0)),
            pl.BlockSpec((IN_DIM, IN_DIM), full),
            pl.BlockSpec((IN_DIM, 512), full),
            pl.BlockSpec((1, 512), full),
            pl.BlockSpec((512, 256), full),
            pl.BlockSpec((1, 256), full),
            pl.BlockSpec((256, 128), full),
            pl.BlockSpec((1, 128), full),
            pl.BlockSpec((128, 1), full),
            pl.BlockSpec((1, 1), full),
        ],
        out_specs=pl.BlockSpec((_BB, 1), lambda i: (i, 0)),
        out_shape=jax.ShapeDtypeStruct((B, 1), jnp.float32),
    )(c, wk, W1, b1, W2, b2, W3, b3, W4, b4)


def kernel(x, tables, field_cov_W, W1, b1, W2, b2, W3, b3, W4, b4):
    idx_raw = x.astype(jnp.int32).reshape(-1)
    t3 = jnp.transpose(tables, (0, 2, 1))
    gathered = _make_sc_gather()(idx_raw, t3)
    c = gathered.reshape(B, IN_DIM)

    w_sym = (field_cov_W + field_cov_W.T) * 0.5
    w0 = w_sym * (1.0 - jnp.eye(F_FIELDS, dtype=jnp.float32))
    wk = (w0[:, None, :, None]
          * jnp.eye(D, dtype=jnp.float32)[None, :, None, :]
          ).reshape(IN_DIM, IN_DIM)

    out = _tc_call(c, wk, W1, b1.reshape(1, -1), W2, b2.reshape(1, -1),
                   W3, b3.reshape(1, -1), W4, b4.reshape(1, -1))
    return out.reshape(B)


# speedup vs baseline: 4.2860x; 1.2638x over previous
import functools

import jax
import jax.numpy as jnp
from jax import lax
from jax.experimental import pallas as pl
from jax.experimental.pallas import tpu as pltpu
from jax.experimental.pallas import tpu_sc as plsc

F_FIELDS = 26
VOCAB = 100000
D = 64
B = 1024
IN_DIM = F_FIELDS * D
N_ROWS = B * F_FIELDS

_NC = 2
_NS = 16
_NW = _NC * _NS
_RPW = N_ROWS // _NW
_CHUNK = 104
_NCHUNK = _RPW // _CHUNK

_WIN = 13
_NGRP = _RPW // _WIN
_OBUF = 4
_GBYTES = _WIN * D * 4


def _sc_gather_body(idx_hbm, t3_hbm, out_hbm, idx_v, outbuf, blks,
                    sem_out, *sems):
    wid = lax.axis_index("s") * _NC + lax.axis_index("c")
    base = wid * _RPW

    pltpu.sync_copy(idx_hbm.at[pl.ds(base, _RPW)], idx_v.at[pl.ds(0, _RPW)])

    def _fetch(j, w):
        v = idx_v[pl.ds(j, 16)][0]
        f = lax.rem(base + j, F_FIELDS)
        voff = pl.multiple_of((v >> 7) << 7, 128)
        pltpu.make_async_copy(
            t3_hbm.at[f, :, pl.ds(voff, 128)], blks.at[w], sems[w]).start()

    for w in range(_WIN):
        _fetch(w, w)

    def _group(g, carry):
        jb = g * _WIN
        obase = lax.rem(g, _OBUF) * (_WIN * D)
        for w in range(_WIN):
            j = jb + w
            pltpu.make_async_copy(
                t3_hbm.at[0, :, pl.ds(0, 128)], blks.at[w], sems[w]).wait()
            v = idx_v[pl.ds(j, 16)][0]
            lane = jnp.full((16,), v & 127, dtype=jnp.int32)
            for k in range(D // 16):
                d_idx = lax.iota(jnp.int32, 16) + (16 * k)
                col = plsc.load_gather(blks.at[w], [d_idx, lane])
                outbuf[pl.ds(obase + w * D + 16 * k, 16)] = col

            @pl.when(g < _NGRP - 1)
            def _():
                _fetch(j + _WIN, w)

        @pl.when(g >= _OBUF - 1)
        def _():
            pltpu.make_async_copy(
                out_hbm.at[pl.ds(0, _WIN * D)],
                outbuf.at[pl.ds(0, _WIN * D)], sem_out).wait()

        pltpu.make_async_copy(
            outbuf.at[pl.ds(obase, _WIN * D)],
            out_hbm.at[pl.ds((base + jb) * D, _WIN * D)], sem_out).start()
        return carry

    lax.fori_loop(0, _NGRP, _group, 0)

    for _ in range(_OBUF - 1):
        pltpu.make_async_copy(
            out_hbm.at[pl.ds(0, _WIN * D)],
            outbuf.at[pl.ds(0, _WIN * D)], sem_out).wait()


@functools.lru_cache(maxsize=1)
def _make_sc_gather():
    mesh = plsc.VectorSubcoreMesh(
        core_axis_name="c", subcore_axis_name="s", num_cores=_NC)
    return pl.kernel(
        _sc_gather_body,
        mesh=mesh,
        compiler_params=pltpu.CompilerParams(needs_layout_passes=False),
        out_type=jax.ShapeDtypeStruct((N_ROWS * D,), jnp.float32),
        scratch_types=[
            pltpu.VMEM((_RPW + 16,), jnp.int32),
            pltpu.VMEM((_OBUF * _WIN * D,), jnp.float32),
            pltpu.VMEM((_WIN, D, 128), jnp.float32),
            pltpu.SemaphoreType.DMA,
        ] + [pltpu.SemaphoreType.DMA] * _WIN,
    )


def _tc_body(c_ref, wk_ref, w1_ref, b1_ref, w2_ref, b2_ref, w3_ref, b3_ref,
             w4_ref, b4_ref, o_ref):
    c = c_ref[...]
    z = jnp.dot(c, wk_ref[...], preferred_element_type=jnp.float32)
    fw = 0.5 * jnp.sum(c * z, axis=1, keepdims=True)
    h = jnp.maximum(
        jnp.dot(c, w1_ref[...], preferred_element_type=jnp.float32)
        + b1_ref[...], 0.0)
    h = jnp.maximum(
        jnp.dot(h, w2_ref[...], preferred_element_type=jnp.float32)
        + b2_ref[...], 0.0)
    h = jnp.maximum(
        jnp.dot(h, w3_ref[...], preferred_element_type=jnp.float32)
        + b3_ref[...], 0.0)
    m = jnp.dot(h, w4_ref[...], preferred_element_type=jnp.float32) + b4_ref[...]
    o_ref[...] = jax.nn.sigmoid(fw + m)


_BB = 256


def _tc_call(c, wk, W1, b1, W2, b2, W3, b3, W4, b4):
    full = lambda i: (0, 0)
    return pl.pallas_call(
        _tc_body,
        grid=(B // _BB,),
        in_specs=[
            pl.BlockSpec((_BB, IN_DIM), lambda i: (i, 0)),
            pl.BlockSpec((IN_DIM, IN_DIM), full),
            pl.BlockSpec((IN_DIM, 512), full),
            pl.BlockSpec((1, 512), full),
            pl.BlockSpec((512, 256), full),
            pl.BlockSpec((1, 256), full),
            pl.BlockSpec((256, 128), full),
            pl.BlockSpec((1, 128), full),
            pl.BlockSpec((128, 1), full),
            pl.BlockSpec((1, 1), full),
        ],
        out_specs=pl.BlockSpec((_BB, 1), lambda i: (i, 0)),
        out_shape=jax.ShapeDtypeStruct((B, 1), jnp.float32),
    )(c, wk, W1, b1, W2, b2, W3, b3, W4, b4)


def kernel(x, tables, field_cov_W, W1, b1, W2, b2, W3, b3, W4, b4):
    idx_raw = x.astype(jnp.int32).reshape(-1)
    t3 = jnp.transpose(tables, (0, 2, 1))
    gathered = _make_sc_gather()(idx_raw, t3)
    c = gathered.reshape(B, IN_DIM)

    w_sym = (field_cov_W + field_cov_W.T) * 0.5
    w0 = w_sym * (1.0 - jnp.eye(F_FIELDS, dtype=jnp.float32))
    wk = (w0[:, None, :, None]
          * jnp.eye(D, dtype=jnp.float32)[None, :, None, :]
          ).reshape(IN_DIM, IN_DIM)

    out = _tc_call(c, wk, W1, b1.reshape(1, -1), W2, b2.reshape(1, -1),
                   W3, b3.reshape(1, -1), W4, b4.reshape(1, -1))
    return out.reshape(B)
